# baseline (speedup 1.0000x reference)
import functools
import jax
import jax.numpy as jnp
from jax import lax
from jax.experimental import pallas as pl
from jax.experimental.pallas import tpu as pltpu
from jax.experimental.pallas import tpu_sc as plsc

H = 768
NC, NS = 2, 16
NW = NC * NS
K = 32
NBUF = 4
EPS = 1e-12


def _sc_gather_body(ntok, ids_hbm, word_hbm, out_hbm, bufs, idxall, gsem,
                    osem):
    wid = lax.axis_index("c") * NS + lax.axis_index("s")
    tok0 = wid * ntok
    nchunk = ntok // K

    pltpu.sync_copy(ids_hbm.at[pl.ds(tok0, ntok)], idxall)

    def issue_gather(c, s):
        pltpu.async_copy(word_hbm.at[idxall.at[pl.ds(c * K, K)]], bufs.at[s],
                         gsem.at[s])

    def wait_gather(s):
        pltpu.make_async_copy(word_hbm.at[pl.ds(0, K)], bufs.at[s],
                              gsem.at[s]).wait()

    def wait_out(s):
        pltpu.make_async_copy(bufs.at[s], out_hbm.at[pl.ds(0, K)],
                              osem.at[s]).wait()

    issue_gather(0, 0)
    issue_gather(1, 1)

    ngroup = nchunk // NBUF

    def group_body(g, carry):
        for k in range(NBUF):
            c = g * NBUF + k
            if k < 2:
                s2 = k + 2

                @pl.when(g > 0)
                def _():
                    wait_out(s2)

                issue_gather(c + 2, s2)
            else:
                s2 = k - 2

                @pl.when(g < ngroup - 1)
                def _():
                    wait_out(s2)
                    issue_gather(c + 2, s2)

            wait_gather(k)
            pltpu.async_copy(bufs.at[k], out_hbm.at[pl.ds(tok0 + c * K, K)],
                             osem.at[k])
        return carry

    lax.fori_loop(0, ngroup, group_body, 0, unroll=False)

    for s in range(NBUF):
        wait_out(s)


def _sc_gather(ids, word_emb):
    n = ids.shape[0]
    ntok = n // NW
    mesh = plsc.VectorSubcoreMesh(core_axis_name="c", subcore_axis_name="s",
                                  num_cores=NC, num_subcores=NS)
    return pl.kernel(
        functools.partial(_sc_gather_body, ntok),
        out_type=jax.ShapeDtypeStruct((n, H), jnp.float32),
        mesh=mesh,
        compiler_params=pltpu.CompilerParams(needs_layout_passes=False,
                                             use_tc_tiling_on_sc=False),
        scratch_types=[
            pltpu.VMEM((NBUF, K, H), jnp.float32),
            pltpu.VMEM((ntok,), jnp.int32),
            pltpu.SemaphoreType.DMA((NBUF,)),
            pltpu.SemaphoreType.DMA((NBUF,)),
        ],
    )(ids, word_emb)


def _tc_ln_body(g_ref, tt_ref, pos_ref, type_ref, w_ref, b_ref, j_ref, o_ref):
    tsel = jnp.where(tt_ref[0] == 1,
                     type_ref[1, :][None, :], type_ref[0, :][None, :])
    x = g_ref[0] + pos_ref[...] + tsel
    jm = j_ref[...]
    mu = jnp.dot(x, jm, preferred_element_type=jnp.float32)
    q = jnp.dot(x * x, jm, preferred_element_type=jnp.float32)
    r = lax.rsqrt(q - mu * mu + EPS)
    y = (x - mu) * r * w_ref[0][None, :] + b_ref[0][None, :]
    o_ref[0] = y


def _tc_ln(gath, tts, pos_emb, type_emb, ln_w, ln_b):
    b, s = tts.shape
    g3 = gath.reshape(b, s, H)
    tt3 = tts.reshape(b, s, 1)
    jm = jnp.full((H, H), 1.0 / H, jnp.float32)
    return pl.pallas_call(
        _tc_ln_body,
        grid=(b,),
        in_specs=[
            pl.BlockSpec((1, s, H), lambda i: (i, 0, 0)),
            pl.BlockSpec((1, s, 1), lambda i: (i, 0, 0)),
            pl.BlockSpec((s, H), lambda i: (0, 0)),
            pl.BlockSpec((2, H), lambda i: (0, 0)),
            pl.BlockSpec((1, H), lambda i: (0, 0)),
            pl.BlockSpec((1, H), lambda i: (0, 0)),
            pl.BlockSpec((H, H), lambda i: (0, 0)),
        ],
        out_specs=pl.BlockSpec((1, s, H), lambda i: (i, 0, 0)),
        out_shape=jax.ShapeDtypeStruct((b, s, H), jnp.float32),
    )(g3, tt3, pos_emb, type_emb, ln_w.reshape(1, H), ln_b.reshape(1, H), jm)


def kernel(input_ids, token_type_ids, word_emb, pos_emb, type_emb, ln_w, ln_b):
    b, s = input_ids.shape
    assert word_emb.shape[1] == H
    ids = input_ids.reshape(-1).astype(jnp.int32)
    tts = token_type_ids.astype(jnp.int32)
    gath = _sc_gather(ids, word_emb)
    return _tc_ln(gath, tts, pos_emb, type_emb, ln_w, ln_b)


# speedup vs baseline: 1.0015x; 1.0015x over previous
import functools
import jax
import jax.numpy as jnp
from jax import lax
from jax.experimental import pallas as pl
from jax.experimental.pallas import tpu as pltpu
from jax.experimental.pallas import tpu_sc as plsc

H = 768
NC, NS = 2, 16
NW = NC * NS
K = 32
NBUF = 4
EPS = 1e-12


def _sc_gather_body(ntok, ids_hbm, word_hbm, out_hbm, bufs, idxall, gsem,
                    osem):
    wid = lax.axis_index("c") * NS + lax.axis_index("s")
    tok0 = wid * ntok
    nchunk = ntok // K

    pltpu.sync_copy(ids_hbm.at[pl.ds(tok0, ntok)], idxall)

    def issue_gather(c, s):
        pltpu.async_copy(word_hbm.at[idxall.at[pl.ds(c * K, K)]], bufs.at[s],
                         gsem.at[s])

    def wait_gather(s):
        pltpu.make_async_copy(word_hbm.at[pl.ds(0, K)], bufs.at[s],
                              gsem.at[s]).wait()

    def wait_out(s):
        pltpu.make_async_copy(bufs.at[s], out_hbm.at[pl.ds(0, K)],
                              osem.at[s]).wait()

    issue_gather(0, 0)
    issue_gather(1, 1)

    ngroup = nchunk // NBUF

    def group_body(g, carry):
        for k in range(NBUF):
            c = g * NBUF + k
            if k < 2:
                s2 = k + 2

                @pl.when(g > 0)
                def _():
                    wait_out(s2)

                issue_gather(c + 2, s2)
            else:
                s2 = k - 2

                @pl.when(g < ngroup - 1)
                def _():
                    wait_out(s2)
                    issue_gather(c + 2, s2)

            wait_gather(k)
            pltpu.async_copy(bufs.at[k], out_hbm.at[pl.ds(tok0 + c * K, K)],
                             osem.at[k])
        return carry

    lax.fori_loop(0, ngroup, group_body, 0, unroll=False)

    for s in range(NBUF):
        wait_out(s)


def _sc_gather(ids, word_emb):
    n = ids.shape[0]
    ntok = n // NW
    mesh = plsc.VectorSubcoreMesh(core_axis_name="c", subcore_axis_name="s",
                                  num_cores=NC, num_subcores=NS)
    return pl.kernel(
        functools.partial(_sc_gather_body, ntok),
        out_type=jax.ShapeDtypeStruct((n, H), jnp.float32),
        mesh=mesh,
        compiler_params=pltpu.CompilerParams(needs_layout_passes=False,
                                             use_tc_tiling_on_sc=False),
        scratch_types=[
            pltpu.VMEM((NBUF, K, H), jnp.float32),
            pltpu.VMEM((ntok,), jnp.int32),
            pltpu.SemaphoreType.DMA((NBUF,)),
            pltpu.SemaphoreType.DMA((NBUF,)),
        ],
    )(ids, word_emb)


def _tc_ln_body(g_ref, tt_ref, pos_ref, td_ref, w_ref, b_ref, j_ref, o_ref):
    x = g_ref[0] + pos_ref[...] + tt_ref[0] * td_ref[...]
    jm = j_ref[...]
    mu = jnp.dot(x.astype(jnp.bfloat16), jm,
                 preferred_element_type=jnp.float32)
    q = jnp.dot((x * x).astype(jnp.bfloat16), jm,
                preferred_element_type=jnp.float32)
    r = lax.rsqrt(q - mu * mu + EPS)
    y = (x - mu) * r * w_ref[0][None, :] + b_ref[0][None, :]
    o_ref[0] = y


def _tc_ln(gath, tts, pos_emb, type_emb, ln_w, ln_b):
    b, s = tts.shape
    g3 = gath.reshape(b, s, H)
    ttf = tts.astype(jnp.float32).reshape(b, s, 1)
    pos2 = pos_emb + type_emb[0][None, :]
    td = (type_emb[1] - type_emb[0]).reshape(1, H)
    jm = jnp.full((H, H), 1.0 / H, jnp.bfloat16)
    return pl.pallas_call(
        _tc_ln_body,
        grid=(b,),
        in_specs=[
            pl.BlockSpec((1, s, H), lambda i: (i, 0, 0)),
            pl.BlockSpec((1, s, 1), lambda i: (i, 0, 0)),
            pl.BlockSpec((s, H), lambda i: (0, 0)),
            pl.BlockSpec((1, H), lambda i: (0, 0)),
            pl.BlockSpec((1, H), lambda i: (0, 0)),
            pl.BlockSpec((1, H), lambda i: (0, 0)),
            pl.BlockSpec((H, H), lambda i: (0, 0)),
        ],
        out_specs=pl.BlockSpec((1, s, H), lambda i: (i, 0, 0)),
        out_shape=jax.ShapeDtypeStruct((b, s, H), jnp.float32),
    )(g3, ttf, pos2, td, ln_w.reshape(1, H), ln_b.reshape(1, H), jm)


def kernel(input_ids, token_type_ids, word_emb, pos_emb, type_emb, ln_w, ln_b):
    b, s = input_ids.shape
    assert word_emb.shape[1] == H
    ids = input_ids.reshape(-1).astype(jnp.int32)
    tts = token_type_ids.astype(jnp.int32)
    gath = _sc_gather(ids, word_emb)
    return _tc_ln(gath, tts, pos_emb, type_emb, ln_w, ln_b)


# speedup vs baseline: 1.0469x; 1.0453x over previous
import functools
import jax
import jax.numpy as jnp
from jax import lax
from jax.experimental import pallas as pl
from jax.experimental.pallas import tpu as pltpu
from jax.experimental.pallas import tpu_sc as plsc

H = 768
NC, NS = 2, 16
NW = NC * NS
K = 32
NBUF = 4
EPS = 1e-12


def _sc_gather_body(ntok, ids_hbm, word_hbm, out_hbm, bufs, idxall, gsem,
                    osem):
    wid = lax.axis_index("c") * NS + lax.axis_index("s")
    tok0 = wid * ntok
    nchunk = ntok // K

    pltpu.sync_copy(ids_hbm.at[pl.ds(tok0, ntok)], idxall)

    def issue_gather(c, s):
        pltpu.async_copy(word_hbm.at[idxall.at[pl.ds(c * K, K)]], bufs.at[s],
                         gsem.at[s])

    def wait_gather(s):
        pltpu.make_async_copy(word_hbm.at[pl.ds(0, K)], bufs.at[s],
                              gsem.at[s]).wait()

    def wait_out(s):
        pltpu.make_async_copy(bufs.at[s], out_hbm.at[pl.ds(0, K)],
                              osem.at[s]).wait()

    issue_gather(0, 0)
    issue_gather(1, 1)

    ngroup = nchunk // NBUF

    def group_body(g, carry):
        for k in range(NBUF):
            c = g * NBUF + k
            if k < 2:
                s2 = k + 2

                @pl.when(g > 0)
                def _():
                    wait_out(s2)

                issue_gather(c + 2, s2)
            else:
                s2 = k - 2

                @pl.when(g < ngroup - 1)
                def _():
                    wait_out(s2)
                    issue_gather(c + 2, s2)

            wait_gather(k)
            pltpu.async_copy(bufs.at[k], out_hbm.at[pl.ds(tok0 + c * K, K)],
                             osem.at[k])
        return carry

    lax.fori_loop(0, ngroup, group_body, 0, unroll=False)

    for s in range(NBUF):
        wait_out(s)


def _sc_gather(ids, word_emb):
    n = ids.shape[0]
    ntok = n // NW
    mesh = plsc.VectorSubcoreMesh(core_axis_name="c", subcore_axis_name="s",
                                  num_cores=NC, num_subcores=NS)
    return pl.kernel(
        functools.partial(_sc_gather_body, ntok),
        out_type=jax.ShapeDtypeStruct((n, H), jnp.float32),
        mesh=mesh,
        compiler_params=pltpu.CompilerParams(needs_layout_passes=False,
                                             use_tc_tiling_on_sc=False),
        scratch_types=[
            pltpu.VMEM((NBUF, K, H), jnp.float32),
            pltpu.VMEM((ntok,), jnp.int32),
            pltpu.SemaphoreType.DMA((NBUF,)),
            pltpu.SemaphoreType.DMA((NBUF,)),
        ],
    )(ids, word_emb)


def _tc_ln_body(g_ref, tt_ref, pos_ref, td_ref, w_ref, b_ref, j_ref, o_ref):
    nb = g_ref.shape[0]
    s, h = pos_ref.shape
    x = (g_ref[...].reshape(nb * s, h)
         + jnp.tile(pos_ref[...], (nb, 1))
         + tt_ref[...].reshape(nb * s, 1) * td_ref[...])
    jm = j_ref[...]
    mu = jnp.dot(x.astype(jnp.bfloat16), jm,
                 preferred_element_type=jnp.float32)
    q = jnp.dot((x * x).astype(jnp.bfloat16), jm,
                preferred_element_type=jnp.float32)
    r = lax.rsqrt(q - mu * mu + EPS)
    y = (x - mu) * r * w_ref[0][None, :] + b_ref[0][None, :]
    o_ref[...] = y.reshape(nb, s, h)


def _tc_ln(gath, tts, pos_emb, type_emb, ln_w, ln_b):
    b, s = tts.shape
    g3 = gath.reshape(b, s, H)
    ttf = tts.astype(jnp.float32).reshape(b, s, 1)
    pos2 = pos_emb + type_emb[0][None, :]
    td = (type_emb[1] - type_emb[0]).reshape(1, H)
    jm = jnp.full((H, H), 1.0 / H, jnp.bfloat16)
    bb = 4
    return pl.pallas_call(
        _tc_ln_body,
        grid=(b // bb,),
        in_specs=[
            pl.BlockSpec((bb, s, H), lambda i: (i, 0, 0)),
            pl.BlockSpec((bb, s, 1), lambda i: (i, 0, 0)),
            pl.BlockSpec((s, H), lambda i: (0, 0)),
            pl.BlockSpec((1, H), lambda i: (0, 0)),
            pl.BlockSpec((1, H), lambda i: (0, 0)),
            pl.BlockSpec((1, H), lambda i: (0, 0)),
            pl.BlockSpec((H, H), lambda i: (0, 0)),
        ],
        out_specs=pl.BlockSpec((bb, s, H), lambda i: (i, 0, 0)),
        out_shape=jax.ShapeDtypeStruct((b, s, H), jnp.float32),
    )(g3, ttf, pos2, td, ln_w.reshape(1, H), ln_b.reshape(1, H), jm)


def kernel(input_ids, token_type_ids, word_emb, pos_emb, type_emb, ln_w, ln_b):
    b, s = input_ids.shape
    assert word_emb.shape[1] == H
    ids = input_ids.reshape(-1).astype(jnp.int32)
    tts = token_type_ids.astype(jnp.int32)
    gath = _sc_gather(ids, word_emb)
    return _tc_ln(gath, tts, pos_emb, type_emb, ln_w, ln_b)


# speedup vs baseline: 1.1896x; 1.1363x over previous
import functools
import jax
import jax.numpy as jnp
from jax import lax
from jax.experimental import pallas as pl
from jax.experimental.pallas import tpu as pltpu
from jax.experimental.pallas import tpu_sc as plsc

H = 768
NC, NS = 2, 16
NW = NC * NS
K = 32
NBUF = 4
EPS = 1e-12


def _sc_gather_body(ntok, ids_hbm, word_hbm, out_hbm, bufs, idxall, gsem,
                    osem):
    wid = lax.axis_index("c") * NS + lax.axis_index("s")
    tok0 = wid * ntok
    nchunk = ntok // K

    pltpu.sync_copy(ids_hbm.at[pl.ds(tok0, ntok)], idxall)

    def issue_gather(c, s):
        pltpu.async_copy(word_hbm.at[idxall.at[pl.ds(c * K, K)]], bufs.at[s],
                         gsem.at[s])

    def wait_gather(s):
        pltpu.make_async_copy(word_hbm.at[pl.ds(0, K)], bufs.at[s],
                              gsem.at[s]).wait()

    def wait_out(s):
        pltpu.make_async_copy(bufs.at[s], out_hbm.at[pl.ds(0, K)],
                              osem.at[s]).wait()

    issue_gather(0, 0)
    issue_gather(1, 1)

    ngroup = nchunk // NBUF

    def group_body(g, carry):
        for k in range(NBUF):
            c = g * NBUF + k
            if k < 2:
                s2 = k + 2

                @pl.when(g > 0)
                def _():
                    wait_out(s2)

                issue_gather(c + 2, s2)
            else:
                s2 = k - 2

                @pl.when(g < ngroup - 1)
                def _():
                    wait_out(s2)
                    issue_gather(c + 2, s2)

            wait_gather(k)
            pltpu.async_copy(bufs.at[k], out_hbm.at[pl.ds(tok0 + c * K, K)],
                             osem.at[k])
        return carry

    lax.fori_loop(0, ngroup, group_body, 0, unroll=False)

    for s in range(NBUF):
        wait_out(s)


def _sc_gather(ids, word_emb):
    n = ids.shape[0]
    ntok = n // NW
    mesh = plsc.VectorSubcoreMesh(core_axis_name="c", subcore_axis_name="s",
                                  num_cores=NC, num_subcores=NS)
    return pl.kernel(
        functools.partial(_sc_gather_body, ntok),
        out_type=jax.ShapeDtypeStruct((n, H), jnp.float32),
        mesh=mesh,
        compiler_params=pltpu.CompilerParams(needs_layout_passes=False,
                                             use_tc_tiling_on_sc=False),
        scratch_types=[
            pltpu.VMEM((NBUF, K, H), jnp.float32),
            pltpu.VMEM((ntok,), jnp.int32),
            pltpu.SemaphoreType.DMA((NBUF,)),
            pltpu.SemaphoreType.DMA((NBUF,)),
        ],
    )(ids, word_emb)


def _tc_ln_body(g_ref, tt_ref, pos_ref, td_ref, w_ref, b_ref, j_ref, o_ref):
    nb = g_ref.shape[0]
    s, h = pos_ref.shape
    x = (g_ref[...].reshape(nb * s, h)
         + jnp.tile(pos_ref[...], (nb, 1))
         + tt_ref[...].reshape(nb * s, 1) * td_ref[...])
    mu = jnp.mean(x, axis=-1, keepdims=True)
    xc = x - mu
    var = jnp.mean(xc * xc, axis=-1, keepdims=True)
    y = xc * lax.rsqrt(var + EPS) * w_ref[0][None, :] + b_ref[0][None, :]
    o_ref[...] = y.reshape(nb, s, h)


def _tc_ln(gath, tts, pos_emb, type_emb, ln_w, ln_b):
    b, s = tts.shape
    g3 = gath.reshape(b, s, H)
    ttf = tts.astype(jnp.float32).reshape(b, s, 1)
    pos2 = pos_emb + type_emb[0][None, :]
    td = (type_emb[1] - type_emb[0]).reshape(1, H)
    jm = jnp.full((H, H), 1.0 / H, jnp.bfloat16)
    bb = 4
    return pl.pallas_call(
        _tc_ln_body,
        grid=(b // bb,),
        in_specs=[
            pl.BlockSpec((bb, s, H), lambda i: (i, 0, 0)),
            pl.BlockSpec((bb, s, 1), lambda i: (i, 0, 0)),
            pl.BlockSpec((s, H), lambda i: (0, 0)),
            pl.BlockSpec((1, H), lambda i: (0, 0)),
            pl.BlockSpec((1, H), lambda i: (0, 0)),
            pl.BlockSpec((1, H), lambda i: (0, 0)),
            pl.BlockSpec((H, H), lambda i: (0, 0)),
        ],
        out_specs=pl.BlockSpec((bb, s, H), lambda i: (i, 0, 0)),
        out_shape=jax.ShapeDtypeStruct((b, s, H), jnp.float32),
    )(g3, ttf, pos2, td, ln_w.reshape(1, H), ln_b.reshape(1, H), jm)


def kernel(input_ids, token_type_ids, word_emb, pos_emb, type_emb, ln_w, ln_b):
    b, s = input_ids.shape
    assert word_emb.shape[1] == H
    ids = input_ids.reshape(-1).astype(jnp.int32)
    tts = token_type_ids.astype(jnp.int32)
    gath = _sc_gather(ids, word_emb)
    return _tc_ln(gath, tts, pos_emb, type_emb, ln_w, ln_b)


# speedup vs baseline: 1.2249x; 1.0297x over previous
import functools
import jax
import jax.numpy as jnp
from jax import lax
from jax.experimental import pallas as pl
from jax.experimental.pallas import tpu as pltpu
from jax.experimental.pallas import tpu_sc as plsc

H = 768
NC, NS = 2, 16
NW = NC * NS
K = 32
NBUF = 4
EPS = 1e-12


def _sc_gather_body(ntok, ids_hbm, word_hbm, out_hbm, bufs, idxall, gsem,
                    osem):
    wid = lax.axis_index("c") * NS + lax.axis_index("s")
    tok0 = wid * ntok
    nchunk = ntok // K

    pltpu.sync_copy(ids_hbm.at[pl.ds(tok0, ntok)], idxall)

    def issue_gather(c, s):
        pltpu.async_copy(word_hbm.at[idxall.at[pl.ds(c * K, K)]], bufs.at[s],
                         gsem.at[s])

    def wait_gather(s):
        pltpu.make_async_copy(word_hbm.at[pl.ds(0, K)], bufs.at[s],
                              gsem.at[s]).wait()

    def wait_out(s):
        pltpu.make_async_copy(bufs.at[s], out_hbm.at[pl.ds(0, K)],
                              osem.at[s]).wait()

    issue_gather(0, 0)
    issue_gather(1, 1)

    ngroup = nchunk // NBUF

    def group_body(g, carry):
        for k in range(NBUF):
            c = g * NBUF + k
            if k < 2:
                s2 = k + 2

                @pl.when(g > 0)
                def _():
                    wait_out(s2)

                issue_gather(c + 2, s2)
            else:
                s2 = k - 2

                @pl.when(g < ngroup - 1)
                def _():
                    wait_out(s2)
                    issue_gather(c + 2, s2)

            wait_gather(k)
            pltpu.async_copy(bufs.at[k], out_hbm.at[pl.ds(tok0 + c * K, K)],
                             osem.at[k])
        return carry

    lax.fori_loop(0, ngroup, group_body, 0, unroll=False)

    for s in range(NBUF):
        wait_out(s)


def _sc_gather(ids, word_emb):
    n = ids.shape[0]
    ntok = n // NW
    mesh = plsc.VectorSubcoreMesh(core_axis_name="c", subcore_axis_name="s",
                                  num_cores=NC, num_subcores=NS)
    return pl.kernel(
        functools.partial(_sc_gather_body, ntok),
        out_type=jax.ShapeDtypeStruct((n, H), jnp.float32),
        mesh=mesh,
        compiler_params=pltpu.CompilerParams(needs_layout_passes=False,
                                             use_tc_tiling_on_sc=False),
        scratch_types=[
            pltpu.VMEM((NBUF, K, H), jnp.float32),
            pltpu.VMEM((ntok,), jnp.int32),
            pltpu.SemaphoreType.DMA((NBUF,)),
            pltpu.SemaphoreType.DMA((NBUF,)),
        ],
    )(ids, word_emb)


def _tc_ln_body(g_ref, tt_ref, pos_ref, td_ref, w_ref, b_ref, o_ref):
    nb = g_ref.shape[0]
    s, h = pos_ref.shape
    x = (g_ref[...].reshape(nb * s, h)
         + jnp.tile(pos_ref[...], (nb, 1))
         + tt_ref[...].reshape(nb * s, 1) * td_ref[...])
    mu = jnp.mean(x, axis=-1, keepdims=True)
    xc = x - mu
    var = jnp.mean(xc * xc, axis=-1, keepdims=True)
    y = xc * lax.rsqrt(var + EPS) * w_ref[0][None, :] + b_ref[0][None, :]
    o_ref[...] = y.reshape(nb, s, h)


def _tc_ln_piece(prev, gath, ttf, pos2, td, lnw, lnb, b, s, piece, npiece):
    bp = b // npiece
    bb = 4
    pblk = piece * (bp // bb)
    g3 = gath.reshape(bp, s, H)
    args = [g3, ttf, pos2, td, lnw, lnb]
    in_specs = [
        pl.BlockSpec((bb, s, H), lambda i: (i, 0, 0)),
        pl.BlockSpec((bb, s, 1), lambda i: (i, 0, 0)),
        pl.BlockSpec((s, H), lambda i: (0, 0)),
        pl.BlockSpec((1, H), lambda i: (0, 0)),
        pl.BlockSpec((1, H), lambda i: (0, 0)),
        pl.BlockSpec((1, H), lambda i: (0, 0)),
    ]
    kwargs = {}
    body = _tc_ln_body
    if prev is not None:
        args = [prev] + args
        in_specs = [pl.BlockSpec(memory_space=pl.ANY)] + in_specs
        kwargs['input_output_aliases'] = {0: 0}
        body = lambda p_ref, *refs: _tc_ln_body(*refs)
    return pl.pallas_call(
        body,
        grid=(bp // bb,),
        in_specs=in_specs,
        out_specs=pl.BlockSpec((bb, s, H), lambda i: (i + pblk, 0, 0)),
        out_shape=jax.ShapeDtypeStruct((b, s, H), jnp.float32),
        **kwargs,
    )(*args)


def kernel(input_ids, token_type_ids, word_emb, pos_emb, type_emb, ln_w, ln_b):
    b, s = input_ids.shape
    assert word_emb.shape[1] == H
    npiece = 4
    bp = b // npiece
    ids = input_ids.reshape(-1).astype(jnp.int32)
    ttf = token_type_ids.astype(jnp.float32).reshape(b, s, 1)
    pos2 = pos_emb + type_emb[0][None, :]
    td = (type_emb[1] - type_emb[0]).reshape(1, H)
    lnw = ln_w.reshape(1, H)
    lnb = ln_b.reshape(1, H)
    gaths = [_sc_gather(ids[i * bp * s:(i + 1) * bp * s], word_emb)
             for i in range(npiece)]
    out = None
    for i in range(npiece):
        out = _tc_ln_piece(out, gaths[i], ttf[i * bp:(i + 1) * bp], pos2, td,
                           lnw, lnb, b, s, i, npiece)
    return out
